# baseline (speedup 1.0000x reference)
import functools

import jax
import jax.numpy as jnp
from jax import lax
from jax.experimental import pallas as pl
from jax.experimental.pallas import tpu as pltpu
from jax.experimental.pallas import tpu_sc as plsc

N = 10000
E = 320000
C = 128
NEG_SLOPE = 0.2

NC = 2
NS = 16
NW = NC * NS
EW = E // NW
K = 96
EWP = 10176
NCHUNK = EWP // K
NP = N + 16
NA = 10240
RPT = NA // NS

ROW_BLK = 1000
GRID = N // ROW_BLK


def _prep_body(x_ref, w_ref, as_ref, ad_ref,
               h_ref, asrc_ref, adst_ref, m_ref, mx_s):
    i = pl.program_id(0)
    h = jnp.dot(x_ref[...], w_ref[...], preferred_element_type=jnp.float32)
    h_ref[...] = h
    a_s = jnp.sum(h * as_ref[...], axis=1)
    a_d = jnp.sum(h * ad_ref[...], axis=1)
    asrc_ref[...] = a_s.reshape(1, 1, ROW_BLK)
    adst_ref[...] = a_d.reshape(1, 1, ROW_BLK)
    bs = jnp.max(a_s)
    bd = jnp.max(a_d)

    @pl.when(i == 0)
    def _init():
        mx_s[0] = bs
        mx_s[1] = bd

    @pl.when(i > 0)
    def _acc():
        mx_s[0] = jnp.maximum(mx_s[0], bs)
        mx_s[1] = jnp.maximum(mx_s[1], bd)

    mm = mx_s[0] + mx_s[1]
    mm = jnp.where(mm >= 0.0, mm, mm * NEG_SLOPE)
    m_ref[...] = jnp.full((C,), mm, dtype=jnp.float32)


def _prep(x, W, att_s, att_d):
    return pl.pallas_call(
        _prep_body,
        grid=(GRID,),
        in_specs=[
            pl.BlockSpec((ROW_BLK, C), lambda i: (i, 0)),
            pl.BlockSpec((C, C), lambda i: (0, 0)),
            pl.BlockSpec((1, C), lambda i: (0, 0)),
            pl.BlockSpec((1, C), lambda i: (0, 0)),
        ],
        out_specs=[
            pl.BlockSpec((ROW_BLK, C), lambda i: (i, 0)),
            pl.BlockSpec((1, 1, ROW_BLK), lambda i: (i, 0, 0)),
            pl.BlockSpec((1, 1, ROW_BLK), lambda i: (i, 0, 0)),
            pl.BlockSpec((C,), lambda i: (0,)),
        ],
        out_shape=[
            jax.ShapeDtypeStruct((N, C), jnp.float32),
            jax.ShapeDtypeStruct((GRID, 1, ROW_BLK), jnp.float32),
            jax.ShapeDtypeStruct((GRID, 1, ROW_BLK), jnp.float32),
            jax.ShapeDtypeStruct((C,), jnp.float32),
        ],
        scratch_shapes=[pltpu.SMEM((2,), jnp.float32)],
    )(x, W, att_s, att_d)


def _sc_body(h_hbm, asrc_hbm, adst_hbm, m_hbm, eidx_hbm,
             zr_hbm, zv_hbm,
             outp_hbm, denp_hbm,
             asrc_t, adst_t, m_v, ex_a, ex_b,
             rows_a, rows_b, si_a, si_b, di_a, di_b,
             acc, den_sh, ga, gb, sa, sb):
    cid = lax.axis_index("c")
    sid = lax.axis_index("s")
    cbase = (sid * NC + cid) * NCHUNK

    pltpu.sync_copy(asrc_hbm, asrc_t)
    pltpu.sync_copy(adst_hbm, adst_t)
    pltpu.sync_copy(m_hbm.at[pl.ds(0, 16)], m_v)
    pltpu.sync_copy(zr_hbm, acc.at[pl.ds(sid * RPT, RPT)])

    @pl.when(sid == 0)
    def _zero_den():
        pltpu.sync_copy(zv_hbm, den_sh)

    mv = m_v[...]

    plsc.subcore_barrier()

    def fire(ci, sbuf, dbuf, rbuf, gsem):
        pltpu.sync_copy(eidx_hbm.at[cbase + ci, 0], sbuf)
        pltpu.sync_copy(eidx_hbm.at[cbase + ci, 1], dbuf)
        pltpu.async_copy(h_hbm.at[sbuf], rbuf, gsem)

    def wait_g(sbuf, rbuf, gsem):
        pltpu.make_async_copy(h_hbm.at[sbuf], rbuf, gsem).wait()

    def compute(sbuf, dbuf, rbuf, exb):
        def group(g, carry):
            sv = sbuf[pl.ds(g * 16, 16)]
            dv = dbuf[pl.ds(g * 16, 16)]
            a1 = plsc.load_gather(asrc_t, [sv])
            a2 = plsc.load_gather(adst_t, [dv])
            al = a1 + a2
            al = jnp.where(al >= 0.0, al, al * NEG_SLOPE)
            exv = jnp.exp(al - mv)
            exb[pl.ds(g * 16, 16)] = exv
            for e2 in range(16):
                s = exv[e2]
                row = g * 16 + e2
                for j in range(C // 16):
                    rbuf[row, pl.ds(j * 16, 16)] = (
                        rbuf[row, pl.ds(j * 16, 16)] * s)
            return carry

        lax.fori_loop(0, K // 16, group, 0)

    def scat(dbuf, rbuf, exb, ssem):
        pltpu.async_copy(exb, den_sh.at[dbuf], ssem, add=True)
        pltpu.async_copy(rbuf, acc.at[dbuf], ssem, add=True)

    def wait_s(dbuf, rbuf, exb, ssem):
        pltpu.make_async_copy(exb, den_sh.at[dbuf], ssem).wait()
        pltpu.make_async_copy(rbuf, acc.at[dbuf], ssem).wait()

    fire(0, si_a, di_a, rows_a, ga)
    wait_g(si_a, rows_a, ga)
    fire(1, si_b, di_b, rows_b, gb)
    compute(si_a, di_a, rows_a, ex_a)
    scat(di_a, rows_a, ex_a, sa)

    def pair(j, carry):
        cb = 2 * j + 1
        wait_g(si_b, rows_b, gb)
        wait_s(di_a, rows_a, ex_a, sa)
        fire(cb + 1, si_a, di_a, rows_a, ga)
        compute(si_b, di_b, rows_b, ex_b)
        scat(di_b, rows_b, ex_b, sb)
        wait_g(si_a, rows_a, ga)
        wait_s(di_b, rows_b, ex_b, sb)
        fire(jnp.minimum(cb + 2, NCHUNK - 1), si_b, di_b, rows_b, gb)
        compute(si_a, di_a, rows_a, ex_a)
        scat(di_a, rows_a, ex_a, sa)
        return carry

    lax.fori_loop(0, (NCHUNK - 1) // 2, pair, 0)

    wait_g(si_b, rows_b, gb)
    wait_s(di_a, rows_a, ex_a, sa)
    compute(si_b, di_b, rows_b, ex_b)
    scat(di_b, rows_b, ex_b, sb)
    wait_s(di_b, rows_b, ex_b, sb)

    plsc.subcore_barrier()

    pltpu.sync_copy(acc.at[pl.ds(sid * RPT, RPT)],
                    outp_hbm.at[cid, pl.ds(sid * RPT, RPT)])

    @pl.when(sid == 0)
    def _den_out():
        pltpu.sync_copy(den_sh, asrc_t)
        pltpu.sync_copy(asrc_t, denp_hbm.at[cid])


def _edge_pass(h, a_src, a_dst, m, eidx, zr, zv):
    mesh = plsc.VectorSubcoreMesh(core_axis_name="c", subcore_axis_name="s")
    fn = functools.partial(
        pl.kernel,
        mesh=mesh,
        compiler_params=pltpu.CompilerParams(needs_layout_passes=False),
        out_type=[
            jax.ShapeDtypeStruct((NC, NA, C), jnp.float32),
            jax.ShapeDtypeStruct((NC, NP), jnp.float32),
        ],
        scratch_types=[
            pltpu.VMEM((NP,), jnp.float32),
            pltpu.VMEM((NP,), jnp.float32),
            pltpu.VMEM((16,), jnp.float32),
            pltpu.VMEM((K,), jnp.float32),
            pltpu.VMEM((K,), jnp.float32),
            pltpu.VMEM((K, C), jnp.float32),
            pltpu.VMEM((K, C), jnp.float32),
            pltpu.VMEM((K,), jnp.int32),
            pltpu.VMEM((K,), jnp.int32),
            pltpu.VMEM((K,), jnp.int32),
            pltpu.VMEM((K,), jnp.int32),
            pltpu.VMEM_SHARED((NA, C), jnp.float32),
            pltpu.VMEM_SHARED((NP,), jnp.float32),
            pltpu.SemaphoreType.DMA,
            pltpu.SemaphoreType.DMA,
            pltpu.SemaphoreType.DMA,
            pltpu.SemaphoreType.DMA,
        ],
    )(_sc_body)
    return fn(h, a_src, a_dst, m, eidx, zr, zv)


def _finish_body(outp_ref, denp_ref, h_ref, asrc_ref, adst_ref, m_ref, b_ref,
                 o_ref):
    s = outp_ref[0] + outp_ref[1]
    d = jnp.sum(denp_ref[:, 0, 0, :], axis=0)
    mm = jnp.max(m_ref[...])
    al = asrc_ref[0, 0] + adst_ref[0, 0]
    al = jnp.where(al >= 0.0, al, al * NEG_SLOPE)
    exl = jnp.exp(al - mm)
    s = s + exl[:, None] * h_ref[...]
    d = d + exl
    o = s / (d + 1e-16)[:, None] + b_ref[...][None, :]
    mx = jnp.max(o, axis=1, keepdims=True)
    lo = o - mx
    o_ref[...] = lo - jnp.log(jnp.sum(jnp.exp(lo), axis=1, keepdims=True))


def _finish(outp, denp, h, a_src, a_dst, m, bias):
    return pl.pallas_call(
        _finish_body,
        grid=(GRID,),
        in_specs=[
            pl.BlockSpec((NC, ROW_BLK, C), lambda i: (0, i, 0)),
            pl.BlockSpec((NC, 1, 1, ROW_BLK), lambda i: (0, i, 0, 0)),
            pl.BlockSpec((ROW_BLK, C), lambda i: (i, 0)),
            pl.BlockSpec((1, 1, ROW_BLK), lambda i: (i, 0, 0)),
            pl.BlockSpec((1, 1, ROW_BLK), lambda i: (i, 0, 0)),
            pl.BlockSpec((C,), lambda i: (0,)),
            pl.BlockSpec((C,), lambda i: (0,)),
        ],
        out_specs=pl.BlockSpec((ROW_BLK, C), lambda i: (i, 0)),
        out_shape=jax.ShapeDtypeStruct((N, C), jnp.float32),
    )(outp, denp, h, a_src, a_dst, m, bias)


def kernel(x, edge_index, W, att_src, att_dst, bias):
    att_s = att_src.reshape(1, C)
    att_d = att_dst.reshape(1, C)
    h, a_src3, a_dst3, m = _prep(x, W, att_s, att_d)
    a_src = jnp.pad(a_src3.reshape(N), (0, NP - N))
    a_dst = jnp.pad(a_dst3.reshape(N), (0, NP - N))
    src_p = jnp.pad(edge_index[0].reshape(NW, EW), ((0, 0), (0, EWP - EW)))
    dst_p = jnp.pad(edge_index[1].reshape(NW, EW), ((0, 0), (0, EWP - EW)),
                    constant_values=N)
    eidx = jnp.stack(
        [src_p.reshape(NW, NCHUNK, K),
         dst_p.reshape(NW, NCHUNK, K)],
        axis=2).reshape(NW * NCHUNK, 2, K)
    zr = jnp.zeros((RPT, C), jnp.float32)
    zv = jnp.zeros((NP,), jnp.float32)
    outp, denp = _edge_pass(h, a_src, a_dst, m, eidx, zr, zv)
    denp_r = denp[:, :N].reshape(NC, GRID, 1, ROW_BLK)
    return _finish(outp[:, :N, :], denp_r, h, a_src3, a_dst3, m, bias)


# speedup vs baseline: 2.2640x; 2.2640x over previous
import functools

import jax
import jax.numpy as jnp
from jax import lax
from jax.experimental import pallas as pl
from jax.experimental.pallas import tpu as pltpu
from jax.experimental.pallas import tpu_sc as plsc

N = 10000
E = 320000
C = 128
NEG_SLOPE = 0.2

NC = 2
NS = 16
NW = NC * NS
EW = E // NW
K = 80
NCHUNK = EW // K
NA = 10112
RPT = NA // NS

ROW_BLK = 1000
GRID = N // ROW_BLK


def _prep_body(x_ref, w_ref, as_ref, ad_ref,
               h_ref, asrc_ref, adst_ref, m_ref, mx_s):
    i = pl.program_id(0)
    h = jnp.dot(x_ref[...], w_ref[...], preferred_element_type=jnp.float32)
    h_ref[...] = h
    a_s = jnp.sum(h * as_ref[...], axis=1)
    a_d = jnp.sum(h * ad_ref[...], axis=1)
    asrc_ref[...] = a_s.reshape(1, 1, ROW_BLK)
    adst_ref[...] = a_d.reshape(1, 1, ROW_BLK)
    bs = jnp.max(a_s)
    bd = jnp.max(a_d)

    @pl.when(i == 0)
    def _init():
        mx_s[0] = bs
        mx_s[1] = bd

    @pl.when(i > 0)
    def _acc():
        mx_s[0] = jnp.maximum(mx_s[0], bs)
        mx_s[1] = jnp.maximum(mx_s[1], bd)

    mm = mx_s[0] + mx_s[1]
    mm = jnp.where(mm >= 0.0, mm, mm * NEG_SLOPE)
    m_ref[...] = jnp.full((C,), mm, dtype=jnp.float32)


def _prep(x, W, att_s, att_d):
    return pl.pallas_call(
        _prep_body,
        grid=(GRID,),
        in_specs=[
            pl.BlockSpec((ROW_BLK, C), lambda i: (i, 0)),
            pl.BlockSpec((C, C), lambda i: (0, 0)),
            pl.BlockSpec((1, C), lambda i: (0, 0)),
            pl.BlockSpec((1, C), lambda i: (0, 0)),
        ],
        out_specs=[
            pl.BlockSpec((ROW_BLK, C), lambda i: (i, 0)),
            pl.BlockSpec((1, 1, ROW_BLK), lambda i: (i, 0, 0)),
            pl.BlockSpec((1, 1, ROW_BLK), lambda i: (i, 0, 0)),
            pl.BlockSpec((C,), lambda i: (0,)),
        ],
        out_shape=[
            jax.ShapeDtypeStruct((N, C), jnp.float32),
            jax.ShapeDtypeStruct((GRID, 1, ROW_BLK), jnp.float32),
            jax.ShapeDtypeStruct((GRID, 1, ROW_BLK), jnp.float32),
            jax.ShapeDtypeStruct((C,), jnp.float32),
        ],
        scratch_shapes=[pltpu.SMEM((2,), jnp.float32)],
    )(x, W, att_s, att_d)


def _sc_body(h_hbm, asrc_hbm, adst_hbm, m_hbm, eidx_hbm,
             zr_hbm, zv_hbm,
             outp_hbm, denp_hbm,
             asrc_t, adst_t, m_v, ex_a, ex_b,
             rows_a, rows_b, sd0, sd1, sd2, sd3,
             acc, den_sh, ga, gb, sa, sb, i0, i1, i2, i3):
    cid = lax.axis_index("c")
    sid = lax.axis_index("s")
    cbase = (sid * NC + cid) * NCHUNK

    pltpu.sync_copy(asrc_hbm, asrc_t)
    pltpu.sync_copy(adst_hbm, adst_t)
    pltpu.sync_copy(m_hbm.at[pl.ds(0, 16)], m_v)
    pltpu.sync_copy(zr_hbm, acc.at[pl.ds(sid * RPT, RPT)])

    @pl.when(sid == 0)
    def _zero_den():
        pltpu.sync_copy(zv_hbm, den_sh)

    mv = m_v[...]

    plsc.subcore_barrier()

    SD = [(sd0, i0), (sd1, i1), (sd2, i2), (sd3, i3)]
    RW = [(rows_a, ex_a, ga, sa), (rows_b, ex_b, gb, sb)]

    def fire_idx(ci, p):
        sd, isem = SD[p]
        pltpu.async_copy(eidx_hbm.at[cbase + ci], sd, isem)

    def wait_idx(ci, p):
        sd, isem = SD[p]
        pltpu.make_async_copy(eidx_hbm.at[cbase + ci], sd, isem).wait()

    def fire_g(p, x):
        sd, _ = SD[p]
        rbuf, _, gsem, _ = RW[x]
        pltpu.async_copy(h_hbm.at[sd.at[0]], rbuf, gsem)

    def wait_g(p, x):
        sd, _ = SD[p]
        rbuf, _, gsem, _ = RW[x]
        pltpu.make_async_copy(h_hbm.at[sd.at[0]], rbuf, gsem).wait()

    def compute(p, x):
        sd, _ = SD[p]
        rbuf, exb, _, _ = RW[x]

        def group(g, carry):
            sv = sd[0, pl.ds(g * 16, 16)]
            dv = sd[1, pl.ds(g * 16, 16)]
            a1 = plsc.load_gather(asrc_t, [sv])
            a2 = plsc.load_gather(adst_t, [dv])
            al = a1 + a2
            al = jnp.where(al >= 0.0, al, al * NEG_SLOPE)
            exv = jnp.exp(al - mv)
            exb[pl.ds(g * 16, 16)] = exv
            for e2 in range(16):
                s = exv[e2]
                row = g * 16 + e2
                for j in range(C // 16):
                    rbuf[row, pl.ds(j * 16, 16)] = (
                        rbuf[row, pl.ds(j * 16, 16)] * s)
            return carry

        lax.fori_loop(0, K // 16, group, 0)

    def scat(p, x):
        sd, _ = SD[p]
        rbuf, exb, _, ssem = RW[x]
        pltpu.async_copy(exb, den_sh.at[sd.at[1]], ssem, add=True)
        pltpu.async_copy(rbuf, acc.at[sd.at[1]], ssem, add=True)

    def wait_s(p, x):
        sd, _ = SD[p]
        rbuf, exb, _, ssem = RW[x]
        pltpu.make_async_copy(exb, den_sh.at[sd.at[1]], ssem).wait()
        pltpu.make_async_copy(rbuf, acc.at[sd.at[1]], ssem).wait()

    def body(c, p, fire_next_idx=True, fire_next_g=True, first=False):
        x = p % 2
        if not first:
            wait_s((p - 1) % 4, 1 - x)
        if fire_next_g:
            wait_idx(c + 1, (p + 1) % 4)
            fire_g((p + 1) % 4, 1 - x)
        if fire_next_idx:
            fire_idx(c + 2, (p + 2) % 4)
        wait_g(p, x)
        compute(p, x)
        scat(p, x)

    pltpu.sync_copy(eidx_hbm.at[cbase], sd0)
    fire_g(0, 0)
    fire_idx(1, 1)
    body(0, 0, first=True)

    def quad(i, carry):
        c = 4 * i + 1
        body(c + 0, 1)
        body(c + 1, 2)
        body(c + 2, 3)
        body(c + 3, 0)
        return carry

    lax.fori_loop(0, (NCHUNK - 5) // 4, quad, 0)

    body(121, 1)
    body(122, 2)
    body(123, 3, fire_next_idx=False)
    body(124, 0, fire_next_idx=False, fire_next_g=False)
    wait_s(0, 0)

    plsc.subcore_barrier()

    pltpu.sync_copy(acc.at[pl.ds(sid * RPT, RPT)],
                    outp_hbm.at[cid, pl.ds(sid * RPT, RPT)])

    @pl.when(sid == 0)
    def _den_out():
        pltpu.sync_copy(den_sh, asrc_t)
        pltpu.sync_copy(asrc_t, denp_hbm.at[cid])


def _edge_pass(h, a_src, a_dst, m, eidx, zr, zv):
    mesh = plsc.VectorSubcoreMesh(core_axis_name="c", subcore_axis_name="s")
    fn = functools.partial(
        pl.kernel,
        mesh=mesh,
        compiler_params=pltpu.CompilerParams(needs_layout_passes=False),
        out_type=[
            jax.ShapeDtypeStruct((NC, NA, C), jnp.float32),
            jax.ShapeDtypeStruct((NC, N), jnp.float32),
        ],
        scratch_types=[
            pltpu.VMEM((N,), jnp.float32),
            pltpu.VMEM((N,), jnp.float32),
            pltpu.VMEM((16,), jnp.float32),
            pltpu.VMEM((K,), jnp.float32),
            pltpu.VMEM((K,), jnp.float32),
            pltpu.VMEM((K, C), jnp.float32),
            pltpu.VMEM((K, C), jnp.float32),
            pltpu.VMEM((2, K), jnp.int32),
            pltpu.VMEM((2, K), jnp.int32),
            pltpu.VMEM((2, K), jnp.int32),
            pltpu.VMEM((2, K), jnp.int32),
            pltpu.VMEM_SHARED((NA, C), jnp.float32),
            pltpu.VMEM_SHARED((N,), jnp.float32),
            pltpu.SemaphoreType.DMA,
            pltpu.SemaphoreType.DMA,
            pltpu.SemaphoreType.DMA,
            pltpu.SemaphoreType.DMA,
            pltpu.SemaphoreType.DMA,
            pltpu.SemaphoreType.DMA,
            pltpu.SemaphoreType.DMA,
            pltpu.SemaphoreType.DMA,
        ],
    )(_sc_body)
    return fn(h, a_src, a_dst, m, eidx, zr, zv)


def _finish_body(outp_ref, denp_ref, h_ref, asrc_ref, adst_ref, m_ref, b_ref,
                 o_ref):
    s = outp_ref[0] + outp_ref[1]
    d = jnp.sum(denp_ref[:, 0, 0, :], axis=0)
    mm = jnp.max(m_ref[...])
    al = asrc_ref[0, 0] + adst_ref[0, 0]
    al = jnp.where(al >= 0.0, al, al * NEG_SLOPE)
    exl = jnp.exp(al - mm)
    s = s + exl[:, None] * h_ref[...]
    d = d + exl
    o = s / (d + 1e-16)[:, None] + b_ref[...][None, :]
    mx = jnp.max(o, axis=1, keepdims=True)
    lo = o - mx
    o_ref[...] = lo - jnp.log(jnp.sum(jnp.exp(lo), axis=1, keepdims=True))


def _finish(outp, denp, h, a_src, a_dst, m, bias):
    return pl.pallas_call(
        _finish_body,
        grid=(GRID,),
        in_specs=[
            pl.BlockSpec((NC, ROW_BLK, C), lambda i: (0, i, 0)),
            pl.BlockSpec((NC, 1, 1, ROW_BLK), lambda i: (0, i, 0, 0)),
            pl.BlockSpec((ROW_BLK, C), lambda i: (i, 0)),
            pl.BlockSpec((1, 1, ROW_BLK), lambda i: (i, 0, 0)),
            pl.BlockSpec((1, 1, ROW_BLK), lambda i: (i, 0, 0)),
            pl.BlockSpec((C,), lambda i: (0,)),
            pl.BlockSpec((C,), lambda i: (0,)),
        ],
        out_specs=pl.BlockSpec((ROW_BLK, C), lambda i: (i, 0)),
        out_shape=jax.ShapeDtypeStruct((N, C), jnp.float32),
    )(outp, denp, h, a_src, a_dst, m, bias)


def kernel(x, edge_index, W, att_src, att_dst, bias):
    att_s = att_src.reshape(1, C)
    att_d = att_dst.reshape(1, C)
    h, a_src3, a_dst3, m = _prep(x, W, att_s, att_d)
    a_src = a_src3.reshape(N)
    a_dst = a_dst3.reshape(N)
    src = edge_index[0]
    dst = edge_index[1]
    eidx = jnp.stack([src.reshape(E // K, K), dst.reshape(E // K, K)], axis=1)
    zr = jnp.zeros((RPT, C), jnp.float32)
    zv = jnp.zeros((N,), jnp.float32)
    outp, denp = _edge_pass(h, a_src, a_dst, m, eidx, zr, zv)
    denp_r = denp.reshape(NC, GRID, 1, ROW_BLK)
    return _finish(outp[:, :N, :], denp_r, h, a_src3, a_dst3, m, bias)


# speedup vs baseline: 2.3282x; 1.0284x over previous
import functools

import jax
import jax.numpy as jnp
from jax import lax
from jax.experimental import pallas as pl
from jax.experimental.pallas import tpu as pltpu
from jax.experimental.pallas import tpu_sc as plsc

N = 10000
E = 320000
C = 128
NEG_SLOPE = 0.2

NC = 2
NS = 16
NW = NC * NS
EW = E // NW
K = 80
NCHUNK = EW // K
NA = 10112
RPT = NA // NS

ROW_BLK = 1000
GRID = N // ROW_BLK


def _prep_body(x_ref, w_ref, as_ref, ad_ref,
               h_ref, asrc_ref, adst_ref, m_ref, mx_s):
    i = pl.program_id(0)
    h = jnp.dot(x_ref[...], w_ref[...], preferred_element_type=jnp.float32)
    h_ref[...] = h
    a_s = jnp.sum(h * as_ref[...], axis=1)
    a_d = jnp.sum(h * ad_ref[...], axis=1)
    asrc_ref[...] = a_s.reshape(1, 1, ROW_BLK)
    adst_ref[...] = a_d.reshape(1, 1, ROW_BLK)
    bs = jnp.max(a_s)
    bd = jnp.max(a_d)

    @pl.when(i == 0)
    def _init():
        mx_s[0] = bs
        mx_s[1] = bd

    @pl.when(i > 0)
    def _acc():
        mx_s[0] = jnp.maximum(mx_s[0], bs)
        mx_s[1] = jnp.maximum(mx_s[1], bd)

    mm = mx_s[0] + mx_s[1]
    mm = jnp.where(mm >= 0.0, mm, mm * NEG_SLOPE)
    m_ref[...] = jnp.full((C,), mm, dtype=jnp.float32)


def _prep(x, W, att_s, att_d):
    return pl.pallas_call(
        _prep_body,
        grid=(GRID,),
        in_specs=[
            pl.BlockSpec((ROW_BLK, C), lambda i: (i, 0)),
            pl.BlockSpec((C, C), lambda i: (0, 0)),
            pl.BlockSpec((1, C), lambda i: (0, 0)),
            pl.BlockSpec((1, C), lambda i: (0, 0)),
        ],
        out_specs=[
            pl.BlockSpec((ROW_BLK, C), lambda i: (i, 0)),
            pl.BlockSpec((1, 1, ROW_BLK), lambda i: (i, 0, 0)),
            pl.BlockSpec((1, 1, ROW_BLK), lambda i: (i, 0, 0)),
            pl.BlockSpec((C,), lambda i: (0,)),
        ],
        out_shape=[
            jax.ShapeDtypeStruct((N, C), jnp.float32),
            jax.ShapeDtypeStruct((GRID, 1, ROW_BLK), jnp.float32),
            jax.ShapeDtypeStruct((GRID, 1, ROW_BLK), jnp.float32),
            jax.ShapeDtypeStruct((C,), jnp.float32),
        ],
        scratch_shapes=[pltpu.SMEM((2,), jnp.float32)],
    )(x, W, att_s, att_d)


def _sc_body(h_hbm, asrc_hbm, adst_hbm, m_hbm, eidx_hbm,
             zr_hbm, zv_hbm,
             outp_hbm, denp_hbm,
             asrc_t, adst_t, m_v, ex_a, ex_b,
             rows_a, rows_b, sd0, sd1, sd2, sd3,
             acc, den_sh, ga, gb, sa, sb, i0, i1, i2, i3):
    cid = lax.axis_index("c")
    sid = lax.axis_index("s")
    cbase = (sid * NC + cid) * NCHUNK

    pltpu.sync_copy(asrc_hbm, asrc_t)
    pltpu.sync_copy(adst_hbm, adst_t)
    pltpu.sync_copy(m_hbm.at[pl.ds(0, 16)], m_v)
    pltpu.sync_copy(zr_hbm, acc.at[pl.ds(sid * RPT, RPT)])

    @pl.when(sid == 0)
    def _zero_den():
        pltpu.sync_copy(zv_hbm, den_sh)

    mv = m_v[...]

    plsc.subcore_barrier()

    SD = [(sd0, i0), (sd1, i1), (sd2, i2), (sd3, i3)]
    RW = [(rows_a, ex_a, ga, sa), (rows_b, ex_b, gb, sb)]

    def fire_idx(ci, p):
        sd, isem = SD[p]
        pltpu.async_copy(eidx_hbm.at[cbase + ci], sd, isem)

    def wait_idx(ci, p):
        sd, isem = SD[p]
        pltpu.make_async_copy(eidx_hbm.at[cbase + ci], sd, isem).wait()

    def fire_g(p, x):
        sd, _ = SD[p]
        rbuf, _, gsem, _ = RW[x]
        pltpu.async_copy(h_hbm.at[sd.at[0]], rbuf, gsem)

    def wait_g(p, x):
        sd, _ = SD[p]
        rbuf, _, gsem, _ = RW[x]
        pltpu.make_async_copy(h_hbm.at[sd.at[0]], rbuf, gsem).wait()

    def compute(p, x):
        sd, _ = SD[p]
        rbuf, exb, _, _ = RW[x]

        def group(g, carry):
            sv = sd[0, pl.ds(g * 16, 16)]
            dv = sd[1, pl.ds(g * 16, 16)]
            a1 = plsc.load_gather(asrc_t, [sv])
            a2 = plsc.load_gather(adst_t, [dv])
            al = a1 + a2
            al = jnp.where(al >= 0.0, al, al * NEG_SLOPE)
            exv = jnp.exp(al - mv)
            exb[pl.ds(g * 16, 16)] = exv
            for e2 in range(16):
                s = exv[e2]
                row = g * 16 + e2
                for j in range(C // 16):
                    rbuf[row, pl.ds(j * 16, 16)] = (
                        rbuf[row, pl.ds(j * 16, 16)] * s)
            return carry

        lax.fori_loop(0, K // 16, group, 0)

    def scat(p, x):
        sd, _ = SD[p]
        rbuf, exb, _, ssem = RW[x]
        pltpu.async_copy(exb, den_sh.at[sd.at[1]], ssem, add=True)
        pltpu.async_copy(rbuf, acc.at[sd.at[1]], ssem, add=True)

    def wait_s(p, x):
        sd, _ = SD[p]
        rbuf, exb, _, ssem = RW[x]
        pltpu.make_async_copy(exb, den_sh.at[sd.at[1]], ssem).wait()
        pltpu.make_async_copy(rbuf, acc.at[sd.at[1]], ssem).wait()

    def body(c, p, fire_next_idx=True, fire_next_g=True, first=False):
        x = p % 2
        if not first:
            wait_s((p - 1) % 4, 1 - x)
        if fire_next_g:
            wait_idx(c + 1, (p + 1) % 4)
            fire_g((p + 1) % 4, 1 - x)
        if fire_next_idx:
            fire_idx(c + 2, (p + 2) % 4)
        wait_g(p, x)
        compute(p, x)
        scat(p, x)

    pltpu.sync_copy(eidx_hbm.at[cbase], sd0)
    fire_g(0, 0)
    fire_idx(1, 1)
    body(0, 0, first=True)

    def quad(i, carry):
        c = 4 * i + 1
        body(c + 0, 1)
        body(c + 1, 2)
        body(c + 2, 3)
        body(c + 3, 0)
        return carry

    lax.fori_loop(0, (NCHUNK - 5) // 4, quad, 0)

    body(121, 1)
    body(122, 2)
    body(123, 3, fire_next_idx=False)
    body(124, 0, fire_next_idx=False, fire_next_g=False)
    wait_s(0, 0)

    plsc.subcore_barrier()

    pltpu.sync_copy(acc.at[pl.ds(sid * RPT, RPT)],
                    outp_hbm.at[cid, pl.ds(sid * RPT, RPT)])

    @pl.when(sid == 0)
    def _den_out():
        pltpu.sync_copy(den_sh, asrc_t)
        pltpu.sync_copy(asrc_t, denp_hbm.at[cid])


def _edge_pass(h, a_src, a_dst, m, eidx, zr, zv):
    mesh = plsc.VectorSubcoreMesh(core_axis_name="c", subcore_axis_name="s")
    fn = functools.partial(
        pl.kernel,
        mesh=mesh,
        compiler_params=pltpu.CompilerParams(needs_layout_passes=False),
        out_type=[
            jax.ShapeDtypeStruct((NC, NA, C), jnp.float32),
            jax.ShapeDtypeStruct((NC, N), jnp.float32),
        ],
        scratch_types=[
            pltpu.VMEM((N,), jnp.float32),
            pltpu.VMEM((N,), jnp.float32),
            pltpu.VMEM((16,), jnp.float32),
            pltpu.VMEM((K,), jnp.float32),
            pltpu.VMEM((K,), jnp.float32),
            pltpu.VMEM((K, C), jnp.float32),
            pltpu.VMEM((K, C), jnp.float32),
            pltpu.VMEM((2, K), jnp.int32),
            pltpu.VMEM((2, K), jnp.int32),
            pltpu.VMEM((2, K), jnp.int32),
            pltpu.VMEM((2, K), jnp.int32),
            pltpu.VMEM_SHARED((NA, C), jnp.float32),
            pltpu.VMEM_SHARED((N,), jnp.float32),
            pltpu.SemaphoreType.DMA,
            pltpu.SemaphoreType.DMA,
            pltpu.SemaphoreType.DMA,
            pltpu.SemaphoreType.DMA,
            pltpu.SemaphoreType.DMA,
            pltpu.SemaphoreType.DMA,
            pltpu.SemaphoreType.DMA,
            pltpu.SemaphoreType.DMA,
        ],
    )(_sc_body)
    return fn(h, a_src, a_dst, m, eidx, zr, zv)


def _finish_body(outp_ref, denp_ref, h_ref, asrc_ref, adst_ref, m_ref, b_ref,
                 o_ref):
    s = outp_ref[0] + outp_ref[1]
    d = jnp.sum(denp_ref[:, 0, 0, :], axis=0)
    mm = jnp.max(m_ref[...])
    al = asrc_ref[0, 0] + adst_ref[0, 0]
    al = jnp.where(al >= 0.0, al, al * NEG_SLOPE)
    exl = jnp.exp(al - mm)
    s = s + exl[:, None] * h_ref[...]
    d = d + exl
    o = s / (d + 1e-16)[:, None] + b_ref[...][None, :]
    mx = jnp.max(o, axis=1, keepdims=True)
    lo = o - mx
    o_ref[...] = lo - jnp.log(jnp.sum(jnp.exp(lo), axis=1, keepdims=True))


def _finish(outp, denp, h, a_src, a_dst, m, bias):
    return pl.pallas_call(
        _finish_body,
        grid=(GRID,),
        in_specs=[
            pl.BlockSpec((NC, ROW_BLK, C), lambda i: (0, i, 0)),
            pl.BlockSpec((NC, 1, 1, ROW_BLK), lambda i: (0, i, 0, 0)),
            pl.BlockSpec((ROW_BLK, C), lambda i: (i, 0)),
            pl.BlockSpec((1, 1, ROW_BLK), lambda i: (i, 0, 0)),
            pl.BlockSpec((1, 1, ROW_BLK), lambda i: (i, 0, 0)),
            pl.BlockSpec((C,), lambda i: (0,)),
            pl.BlockSpec((C,), lambda i: (0,)),
        ],
        out_specs=pl.BlockSpec((ROW_BLK, C), lambda i: (i, 0)),
        out_shape=jax.ShapeDtypeStruct((N, C), jnp.float32),
    )(outp, denp, h, a_src, a_dst, m, bias)


def kernel(x, edge_index, W, att_src, att_dst, bias):
    att_s = att_src.reshape(1, C)
    att_d = att_dst.reshape(1, C)
    h, a_src3, a_dst3, m = _prep(x, W, att_s, att_d)
    a_src = a_src3.reshape(N)
    a_dst = a_dst3.reshape(N)
    src = edge_index[0]
    dst = edge_index[1]
    eidx = jnp.stack([src.reshape(E // K, K), dst.reshape(E // K, K)], axis=1)
    zr = jnp.zeros((RPT, C), jnp.float32)
    zv = jnp.zeros((N,), jnp.float32)
    outp, denp = _edge_pass(h, a_src, a_dst, m, eidx, zr, zv)
    denp_r = denp.reshape(NC, GRID, 1, ROW_BLK)
    return _finish(outp, denp_r, h, a_src3, a_dst3, m, bias)


# speedup vs baseline: 2.3615x; 1.0143x over previous
import functools

import jax
import jax.numpy as jnp
from jax import lax
from jax.experimental import pallas as pl
from jax.experimental.pallas import tpu as pltpu
from jax.experimental.pallas import tpu_sc as plsc

N = 10000
E = 320000
C = 128
NEG_SLOPE = 0.2

NC = 2
NS = 16
NW = NC * NS
EW = E // NW
K = 80
NCHUNK = EW // K
NA = 10112
RPT = NA // NS

ROW_BLK = 2000
GRID = N // ROW_BLK


def _prep_body(x_ref, w_ref, as_ref, ad_ref,
               h_ref, asrc_ref, adst_ref, m_ref, mx_s):
    i = pl.program_id(0)
    h = jnp.dot(x_ref[...], w_ref[...], preferred_element_type=jnp.float32)
    h_ref[...] = h
    a_s = jnp.sum(h * as_ref[...], axis=1)
    a_d = jnp.sum(h * ad_ref[...], axis=1)
    asrc_ref[...] = a_s.reshape(1, 1, ROW_BLK)
    adst_ref[...] = a_d.reshape(1, 1, ROW_BLK)
    bs = jnp.max(a_s)
    bd = jnp.max(a_d)

    @pl.when(i == 0)
    def _init():
        mx_s[0] = bs
        mx_s[1] = bd

    @pl.when(i > 0)
    def _acc():
        mx_s[0] = jnp.maximum(mx_s[0], bs)
        mx_s[1] = jnp.maximum(mx_s[1], bd)

    mm = mx_s[0] + mx_s[1]
    mm = jnp.where(mm >= 0.0, mm, mm * NEG_SLOPE)
    m_ref[...] = jnp.full((C,), mm, dtype=jnp.float32)


def _prep(x, W, att_s, att_d):
    return pl.pallas_call(
        _prep_body,
        grid=(GRID,),
        in_specs=[
            pl.BlockSpec((ROW_BLK, C), lambda i: (i, 0)),
            pl.BlockSpec((C, C), lambda i: (0, 0)),
            pl.BlockSpec((1, C), lambda i: (0, 0)),
            pl.BlockSpec((1, C), lambda i: (0, 0)),
        ],
        out_specs=[
            pl.BlockSpec((ROW_BLK, C), lambda i: (i, 0)),
            pl.BlockSpec((1, 1, ROW_BLK), lambda i: (i, 0, 0)),
            pl.BlockSpec((1, 1, ROW_BLK), lambda i: (i, 0, 0)),
            pl.BlockSpec((C,), lambda i: (0,)),
        ],
        out_shape=[
            jax.ShapeDtypeStruct((N, C), jnp.float32),
            jax.ShapeDtypeStruct((GRID, 1, ROW_BLK), jnp.float32),
            jax.ShapeDtypeStruct((GRID, 1, ROW_BLK), jnp.float32),
            jax.ShapeDtypeStruct((C,), jnp.float32),
        ],
        scratch_shapes=[pltpu.SMEM((2,), jnp.float32)],
    )(x, W, att_s, att_d)


def _sc_body(h_hbm, asrc_hbm, adst_hbm, m_hbm, eidx_hbm,
             zr_hbm, zv_hbm,
             outp_hbm, denp_hbm,
             asrc_t, adst_t, m_v, ex_a, ex_b,
             rows_a, rows_b, sd0, sd1, sd2, sd3,
             acc, den_sh, ga, gb, sa, sb, i0, i1, i2, i3):
    cid = lax.axis_index("c")
    sid = lax.axis_index("s")
    cbase = (sid * NC + cid) * NCHUNK

    pltpu.sync_copy(asrc_hbm, asrc_t)
    pltpu.sync_copy(adst_hbm, adst_t)
    pltpu.sync_copy(m_hbm.at[pl.ds(0, 16)], m_v)
    pltpu.sync_copy(zr_hbm, acc.at[pl.ds(sid * RPT, RPT)])

    @pl.when(sid == 0)
    def _zero_den():
        pltpu.sync_copy(zv_hbm, den_sh)

    mv = m_v[...]

    plsc.subcore_barrier()

    SD = [(sd0, i0), (sd1, i1), (sd2, i2), (sd3, i3)]
    RW = [(rows_a, ex_a, ga, sa), (rows_b, ex_b, gb, sb)]

    def fire_idx(ci, p):
        sd, isem = SD[p]
        pltpu.async_copy(eidx_hbm.at[cbase + ci], sd, isem)

    def wait_idx(ci, p):
        sd, isem = SD[p]
        pltpu.make_async_copy(eidx_hbm.at[cbase + ci], sd, isem).wait()

    def fire_g(p, x):
        sd, _ = SD[p]
        rbuf, _, gsem, _ = RW[x]
        pltpu.async_copy(h_hbm.at[sd.at[0]], rbuf, gsem)

    def wait_g(p, x):
        sd, _ = SD[p]
        rbuf, _, gsem, _ = RW[x]
        pltpu.make_async_copy(h_hbm.at[sd.at[0]], rbuf, gsem).wait()

    def compute(p, x):
        sd, _ = SD[p]
        rbuf, exb, _, _ = RW[x]

        def group(g, carry):
            sv = sd[0, pl.ds(g * 16, 16)]
            dv = sd[1, pl.ds(g * 16, 16)]
            a1 = plsc.load_gather(asrc_t, [sv])
            a2 = plsc.load_gather(adst_t, [dv])
            al = a1 + a2
            al = jnp.where(al >= 0.0, al, al * NEG_SLOPE)
            exv = jnp.exp(al - mv)
            exb[pl.ds(g * 16, 16)] = exv
            for e2 in range(16):
                s = exv[e2]
                row = g * 16 + e2
                for j in range(C // 16):
                    rbuf[row, pl.ds(j * 16, 16)] = (
                        rbuf[row, pl.ds(j * 16, 16)] * s)
            return carry

        lax.fori_loop(0, K // 16, group, 0)

    def scat(p, x):
        sd, _ = SD[p]
        rbuf, exb, _, ssem = RW[x]
        pltpu.async_copy(exb, den_sh.at[sd.at[1]], ssem, add=True)
        pltpu.async_copy(rbuf, acc.at[sd.at[1]], ssem, add=True)

    def wait_s(p, x):
        sd, _ = SD[p]
        rbuf, exb, _, ssem = RW[x]
        pltpu.make_async_copy(exb, den_sh.at[sd.at[1]], ssem).wait()
        pltpu.make_async_copy(rbuf, acc.at[sd.at[1]], ssem).wait()

    def body(c, p, fire_next_idx=True, fire_next_g=True, first=False):
        x = p % 2
        if not first:
            wait_s((p - 1) % 4, 1 - x)
        if fire_next_g:
            wait_idx(c + 1, (p + 1) % 4)
            fire_g((p + 1) % 4, 1 - x)
        if fire_next_idx:
            fire_idx(c + 2, (p + 2) % 4)
        wait_g(p, x)
        compute(p, x)
        scat(p, x)

    pltpu.sync_copy(eidx_hbm.at[cbase], sd0)
    fire_g(0, 0)
    fire_idx(1, 1)
    body(0, 0, first=True)

    def quad(i, carry):
        c = 4 * i + 1
        body(c + 0, 1)
        body(c + 1, 2)
        body(c + 2, 3)
        body(c + 3, 0)
        return carry

    lax.fori_loop(0, (NCHUNK - 5) // 4, quad, 0)

    body(121, 1)
    body(122, 2)
    body(123, 3, fire_next_idx=False)
    body(124, 0, fire_next_idx=False, fire_next_g=False)
    wait_s(0, 0)

    plsc.subcore_barrier()

    pltpu.sync_copy(acc.at[pl.ds(sid * RPT, RPT)],
                    outp_hbm.at[cid, pl.ds(sid * RPT, RPT)])

    @pl.when(sid == 0)
    def _den_out():
        pltpu.sync_copy(den_sh, asrc_t)
        pltpu.sync_copy(asrc_t, denp_hbm.at[cid])


def _edge_pass(h, a_src, a_dst, m, eidx, zr, zv):
    mesh = plsc.VectorSubcoreMesh(core_axis_name="c", subcore_axis_name="s")
    fn = functools.partial(
        pl.kernel,
        mesh=mesh,
        compiler_params=pltpu.CompilerParams(needs_layout_passes=False),
        out_type=[
            jax.ShapeDtypeStruct((NC, NA, C), jnp.float32),
            jax.ShapeDtypeStruct((NC, N), jnp.float32),
        ],
        scratch_types=[
            pltpu.VMEM((N,), jnp.float32),
            pltpu.VMEM((N,), jnp.float32),
            pltpu.VMEM((16,), jnp.float32),
            pltpu.VMEM((K,), jnp.float32),
            pltpu.VMEM((K,), jnp.float32),
            pltpu.VMEM((K, C), jnp.float32),
            pltpu.VMEM((K, C), jnp.float32),
            pltpu.VMEM((2, K), jnp.int32),
            pltpu.VMEM((2, K), jnp.int32),
            pltpu.VMEM((2, K), jnp.int32),
            pltpu.VMEM((2, K), jnp.int32),
            pltpu.VMEM_SHARED((NA, C), jnp.float32),
            pltpu.VMEM_SHARED((N,), jnp.float32),
            pltpu.SemaphoreType.DMA,
            pltpu.SemaphoreType.DMA,
            pltpu.SemaphoreType.DMA,
            pltpu.SemaphoreType.DMA,
            pltpu.SemaphoreType.DMA,
            pltpu.SemaphoreType.DMA,
            pltpu.SemaphoreType.DMA,
            pltpu.SemaphoreType.DMA,
        ],
    )(_sc_body)
    return fn(h, a_src, a_dst, m, eidx, zr, zv)


def _finish_body(outp_ref, denp_ref, h_ref, asrc_ref, adst_ref, m_ref, b_ref,
                 o_ref):
    s = outp_ref[0] + outp_ref[1]
    d = jnp.sum(denp_ref[:, 0, 0, :], axis=0)
    mm = jnp.max(m_ref[...])
    al = asrc_ref[0, 0] + adst_ref[0, 0]
    al = jnp.where(al >= 0.0, al, al * NEG_SLOPE)
    exl = jnp.exp(al - mm)
    s = s + exl[:, None] * h_ref[...]
    d = d + exl
    o = s / (d + 1e-16)[:, None] + b_ref[...][None, :]
    mx = jnp.max(o, axis=1, keepdims=True)
    lo = o - mx
    o_ref[...] = lo - jnp.log(jnp.sum(jnp.exp(lo), axis=1, keepdims=True))


def _finish(outp, denp, h, a_src, a_dst, m, bias):
    return pl.pallas_call(
        _finish_body,
        grid=(GRID,),
        in_specs=[
            pl.BlockSpec((NC, ROW_BLK, C), lambda i: (0, i, 0)),
            pl.BlockSpec((NC, 1, 1, ROW_BLK), lambda i: (0, i, 0, 0)),
            pl.BlockSpec((ROW_BLK, C), lambda i: (i, 0)),
            pl.BlockSpec((1, 1, ROW_BLK), lambda i: (i, 0, 0)),
            pl.BlockSpec((1, 1, ROW_BLK), lambda i: (i, 0, 0)),
            pl.BlockSpec((C,), lambda i: (0,)),
            pl.BlockSpec((C,), lambda i: (0,)),
        ],
        out_specs=pl.BlockSpec((ROW_BLK, C), lambda i: (i, 0)),
        out_shape=jax.ShapeDtypeStruct((N, C), jnp.float32),
    )(outp, denp, h, a_src, a_dst, m, bias)


def kernel(x, edge_index, W, att_src, att_dst, bias):
    att_s = att_src.reshape(1, C)
    att_d = att_dst.reshape(1, C)
    h, a_src3, a_dst3, m = _prep(x, W, att_s, att_d)
    a_src = a_src3.reshape(N)
    a_dst = a_dst3.reshape(N)
    src = edge_index[0]
    dst = edge_index[1]
    eidx = jnp.stack([src.reshape(E // K, K), dst.reshape(E // K, K)], axis=1)
    zr = jnp.zeros((RPT, C), jnp.float32)
    zv = jnp.zeros((N,), jnp.float32)
    outp, denp = _edge_pass(h, a_src, a_dst, m, eidx, zr, zv)
    denp_r = denp.reshape(NC, GRID, 1, ROW_BLK)
    return _finish(outp, denp_r, h, a_src3, a_dst3, m, bias)
